# baseline (speedup 1.0000x reference)
import jax
import jax.numpy as jnp
from jax.experimental import pallas as pl

_N = 51
_NP = 64
_B = 32
_H = 32
_G4 = 4 * _H
_T = 32
_E = 1024
_EP = 1088


def _sig(x):
    return jax.nn.sigmoid(x)


def _fwd_kernel(ts_ref, src_ref, dst_ref, eps_ref,
                ewx0_ref, ewh0_ref, eb0_ref, ewc1_ref, eb1_ref,
                dwx0_ref, dwh0_ref, db0_ref, dwc1_ref, db1_ref,
                muw_ref, mub_ref, lvw_ref, lvb_ref,
                ow_ref, ob_ref, olvw_ref, olvb_ref,
                out_ref, outlv_ref, lat_ref, mu_ref, lv_ref):
    f32 = jnp.float32

    lane_e = jax.lax.broadcasted_iota(jnp.int32, (_EP, _NP), 1)
    s_onehot = (src_ref[...] == lane_e).astype(f32)
    row_e = jax.lax.broadcasted_iota(jnp.int32, (_NP, _EP), 0)
    d_onehot_t = (dst_ref[...] == row_e).astype(f32)
    cnt = jnp.dot(d_onehot_t, s_onehot, preferred_element_type=f32)
    deg = jnp.sum(cnt, axis=1, keepdims=True)
    dinv = jnp.where(deg > 0.0, jax.lax.rsqrt(deg), 0.0)
    rr = jax.lax.broadcasted_iota(jnp.int32, (_NP, _NP), 0)
    cc = jax.lax.broadcasted_iota(jnp.int32, (_NP, _NP), 1)
    ddiag = jnp.where(rr == cc, dinv, 0.0)
    adj = dinv * jnp.dot(cnt, ddiag, preferred_element_type=f32)

    def a_app(z3):
        return jax.lax.dot_general(adj, z3, (((1,), (0,)), ((), ())),
                                   preferred_element_type=f32)

    def lstm_step(x2, h0, c0, h1, c1, wx0, wh0, b0, wc1, b1):
        zx = x2[:, :, None] * wx0[None]
        zh = jnp.dot(jnp.reshape(h0, (_NP * _B, _H)), wh0,
                     preferred_element_type=f32)
        gates = a_app(zx + jnp.reshape(zh, (_NP, _B, _G4))) + b0[None]
        i = _sig(gates[:, :, 0:_H])
        f = _sig(gates[:, :, _H:2 * _H])
        g = jnp.tanh(gates[:, :, 2 * _H:3 * _H])
        o = _sig(gates[:, :, 3 * _H:])
        c0n = f * c0 + i * g
        h0n = o * jnp.tanh(c0n)
        z1 = jnp.dot(
            jnp.reshape(jnp.concatenate([h0n, h1], axis=2),
                        (_NP * _B, 2 * _H)),
            wc1, preferred_element_type=f32)
        gates1 = a_app(jnp.reshape(z1, (_NP, _B, _G4))) + b1[None]
        i1 = _sig(gates1[:, :, 0:_H])
        f1 = _sig(gates1[:, :, _H:2 * _H])
        g1 = jnp.tanh(gates1[:, :, 2 * _H:3 * _H])
        o1 = _sig(gates1[:, :, 3 * _H:])
        c1n = f1 * c1 + i1 * g1
        h1n = o1 * jnp.tanh(c1n)
        return h0n, c0n, h1n, c1n

    ewx0 = ewx0_ref[...]
    ewh0 = ewh0_ref[...]
    eb0 = eb0_ref[...]
    ewc1 = ewc1_ref[...]
    eb1 = eb1_ref[...]

    def enc_body(t, carry):
        h0, c0, h1, c1 = carry
        x2 = jnp.reshape(ts_ref[pl.ds(t, 1)], (_NP, _B))
        return lstm_step(x2, h0, c0, h1, c1, ewx0, ewh0, eb0, ewc1, eb1)

    z3 = jnp.zeros((_NP, _B, _H), f32)
    h0, c0, h1, c1 = jax.lax.fori_loop(0, _T, enc_body, (z3, z3, z3, z3))

    muw = muw_ref[...]
    lvw = lvw_ref[...]
    mu = (jnp.sum(h1[:, :, :, None] * muw[:, None, :, :], axis=2)
          + mub_ref[...][:, None, :])
    lv = (jnp.sum(h1[:, :, :, None] * lvw[:, None, :, :], axis=2)
          + lvb_ref[...][:, None, :])
    lat = mu + eps_ref[...] * jnp.exp(lv)
    mu_ref[...] = mu
    lv_ref[...] = lv
    lat_ref[...] = lat

    ow = ow_ref[...]
    ob = ob_ref[...]
    olvw = olvw_ref[...]
    olvb = olvb_ref[...]

    def head(h3, w2, b2):
        return jnp.sum(h3 * w2[:, None, :], axis=2) + b2

    o_last = head(lat, ow, ob)
    out_ref[_T - 1] = o_last
    outlv_ref[_T - 1] = head(lat, olvw, olvb)

    dwx0 = dwx0_ref[...]
    dwh0 = dwh0_ref[...]
    db0 = db0_ref[...]
    dwc1 = dwc1_ref[...]
    db1 = db1_ref[...]

    def dec_body(j, carry):
        x2, h0d, c0d, h1d, c1d = carry
        h0n, c0n, h1n, c1n = lstm_step(x2, h0d, c0d, h1d, c1d,
                                       dwx0, dwh0, db0, dwc1, db1)
        o = head(h1n, ow, ob)
        olv = head(h1n, olvw, olvb)
        idx = _T - 2 - j
        out_ref[pl.ds(idx, 1)] = o[None]
        outlv_ref[pl.ds(idx, 1)] = olv[None]
        return o, h0n, c0n, h1n, c1n

    jax.lax.fori_loop(0, _T - 1, dec_body, (o_last, h0, c0, h1, c1))


def _pad_nodes(x, np_=_NP):
    pad = [(0, np_ - x.shape[0])] + [(0, 0)] * (x.ndim - 1)
    return jnp.pad(x, pad)


def kernel(ts_batch, edge_index, params):
    f32 = jnp.float32
    ts_k = jnp.transpose(ts_batch[..., 0], (1, 2, 0))
    ts_k = jnp.pad(ts_k, ((0, 0), (0, _NP - _N), (0, 0)))

    ei = edge_index.astype(jnp.int32)
    loops = jnp.arange(_N, dtype=jnp.int32)
    src_a = jnp.concatenate([ei[0], loops])
    dst_a = jnp.concatenate([ei[1], loops])
    padn = _EP - src_a.shape[0]
    src_col = jnp.pad(src_a, (0, padn), constant_values=-1).reshape(_EP, 1)
    dst_row = jnp.pad(dst_a, (0, padn), constant_values=-1).reshape(1, _EP)

    eps = jax.random.normal(jax.random.key(42), (_B, _N, _H), dtype=f32)
    eps_k = _pad_nodes(jnp.transpose(eps, (1, 0, 2)))

    def layer0_args(p):
        return (p["Wx"].astype(f32), p["Wh"].astype(f32),
                (p["bx"] + p["bh"]).reshape(1, _G4).astype(f32))

    def layer1_args(p):
        wcat = jnp.concatenate([p["Wx"], p["Wh"]], axis=0).astype(f32)
        return wcat, (p["bx"] + p["bh"]).reshape(1, _G4).astype(f32)

    ewx0, ewh0, eb0 = layer0_args(params["enc"][0])
    ewc1, eb1 = layer1_args(params["enc"][1])
    dwx0, dwh0, db0 = layer0_args(params["dec"][0])
    dwc1, db1 = layer1_args(params["dec"][1])

    muw = _pad_nodes(params["mu_W"])
    mub = _pad_nodes(params["mu_b"])
    lvw = _pad_nodes(params["lv_W"])
    lvb = _pad_nodes(params["lv_b"])
    ow = _pad_nodes(params["out_W"][:, :, 0])
    ob = _pad_nodes(params["out_b"])
    olvw = _pad_nodes(params["outlv_W"][:, :, 0])
    olvb = _pad_nodes(params["outlv_b"])

    out_shapes = (
        jax.ShapeDtypeStruct((_T, _NP, _B), f32),
        jax.ShapeDtypeStruct((_T, _NP, _B), f32),
        jax.ShapeDtypeStruct((_NP, _B, _H), f32),
        jax.ShapeDtypeStruct((_NP, _B, _H), f32),
        jax.ShapeDtypeStruct((_NP, _B, _H), f32),
    )
    out_k, outlv_k, lat_k, mu_k, lv_k = pl.pallas_call(
        _fwd_kernel, out_shape=out_shapes)(
        ts_k, src_col, dst_row, eps_k,
        ewx0, ewh0, eb0, ewc1, eb1,
        dwx0, dwh0, db0, dwc1, db1,
        muw, mub, lvw, lvb, ow, ob, olvw, olvb)

    out = jnp.transpose(out_k[:, :_N, :], (2, 0, 1))[..., None]
    outlv = jnp.transpose(outlv_k[:, :_N, :], (2, 0, 1))[..., None]
    lat = jnp.transpose(lat_k[:_N], (1, 0, 2))
    mu = jnp.transpose(mu_k[:_N], (1, 0, 2))
    lv = jnp.transpose(lv_k[:_N], (1, 0, 2))
    return out, lat, mu, lv, outlv


# speedup vs baseline: 1.0381x; 1.0381x over previous
import jax
import jax.numpy as jnp
from jax.experimental import pallas as pl

_N = 51
_NP = 64
_B = 32
_H = 32
_G4 = 4 * _H
_T = 32
_E = 1024
_EP = 1088


def _sig(x):
    return jax.nn.sigmoid(x)


def _fwd_kernel(ts_ref, src_ref, dst_ref, eps_ref,
                ewx0_ref, ewh0_ref, eb0_ref, ewc1_ref, eb1_ref,
                dwx0_ref, dwh0_ref, db0_ref, dwc1_ref, db1_ref,
                muw_ref, mub_ref, lvw_ref, lvb_ref,
                ow_ref, ob_ref, olvw_ref, olvb_ref,
                out_ref, outlv_ref, lat_ref, mu_ref, lv_ref):
    f32 = jnp.float32

    lane_e = jax.lax.broadcasted_iota(jnp.int32, (_EP, _NP), 1)
    s_onehot = (src_ref[...] == lane_e).astype(f32)
    row_e = jax.lax.broadcasted_iota(jnp.int32, (_NP, _EP), 0)
    d_onehot_t = (dst_ref[...] == row_e).astype(f32)
    cnt = jnp.dot(d_onehot_t, s_onehot, preferred_element_type=f32)
    deg = jnp.sum(cnt, axis=1, keepdims=True)
    dinv = jnp.where(deg > 0.0, jax.lax.rsqrt(deg), 0.0)
    rr = jax.lax.broadcasted_iota(jnp.int32, (_NP, _NP), 0)
    cc = jax.lax.broadcasted_iota(jnp.int32, (_NP, _NP), 1)
    ddiag = jnp.where(rr == cc, dinv, 0.0)
    adj = dinv * jnp.dot(cnt, ddiag, preferred_element_type=f32)

    adj16 = adj.astype(jnp.bfloat16)

    def a_app(z3):
        return jax.lax.dot_general(adj16, z3.astype(jnp.bfloat16),
                                   (((1,), (0,)), ((), ())),
                                   preferred_element_type=f32)

    def lstm_step(x2, h0, c0, h1, c1, wx0, wh0, b0, wc1, b1):
        zx = x2[:, :, None] * wx0[None]
        zh = jnp.dot(jnp.reshape(h0, (_NP * _B, _H)).astype(jnp.bfloat16),
                     wh0, preferred_element_type=f32)
        gates = a_app(zx + jnp.reshape(zh, (_NP, _B, _G4))) + b0[None]
        i = _sig(gates[:, :, 0:_H])
        f = _sig(gates[:, :, _H:2 * _H])
        g = jnp.tanh(gates[:, :, 2 * _H:3 * _H])
        o = _sig(gates[:, :, 3 * _H:])
        c0n = f * c0 + i * g
        h0n = o * jnp.tanh(c0n)
        z1 = jnp.dot(
            jnp.reshape(jnp.concatenate([h0n, h1], axis=2),
                        (_NP * _B, 2 * _H)).astype(jnp.bfloat16),
            wc1, preferred_element_type=f32)
        gates1 = a_app(jnp.reshape(z1, (_NP, _B, _G4))) + b1[None]
        i1 = _sig(gates1[:, :, 0:_H])
        f1 = _sig(gates1[:, :, _H:2 * _H])
        g1 = jnp.tanh(gates1[:, :, 2 * _H:3 * _H])
        o1 = _sig(gates1[:, :, 3 * _H:])
        c1n = f1 * c1 + i1 * g1
        h1n = o1 * jnp.tanh(c1n)
        return h0n, c0n, h1n, c1n

    ewx0 = ewx0_ref[...]
    ewh0 = ewh0_ref[...]
    eb0 = eb0_ref[...]
    ewc1 = ewc1_ref[...]
    eb1 = eb1_ref[...]

    def enc_body(t, carry):
        h0, c0, h1, c1 = carry
        x2 = jnp.reshape(ts_ref[pl.ds(t, 1)], (_NP, _B))
        return lstm_step(x2, h0, c0, h1, c1, ewx0, ewh0, eb0, ewc1, eb1)

    z3 = jnp.zeros((_NP, _B, _H), f32)
    h0, c0, h1, c1 = jax.lax.fori_loop(0, _T, enc_body, (z3, z3, z3, z3))

    muw = muw_ref[...]
    lvw = lvw_ref[...]
    mu = (jnp.sum(h1[:, :, :, None] * muw[:, None, :, :], axis=2)
          + mub_ref[...][:, None, :])
    lv = (jnp.sum(h1[:, :, :, None] * lvw[:, None, :, :], axis=2)
          + lvb_ref[...][:, None, :])
    lat = mu + eps_ref[...] * jnp.exp(lv)
    mu_ref[...] = mu
    lv_ref[...] = lv
    lat_ref[...] = lat

    ow = ow_ref[...]
    ob = ob_ref[...]
    olvw = olvw_ref[...]
    olvb = olvb_ref[...]

    def head(h3, w2, b2):
        return jnp.sum(h3 * w2[:, None, :], axis=2) + b2

    o_last = head(lat, ow, ob)
    out_ref[_T - 1] = o_last
    outlv_ref[_T - 1] = head(lat, olvw, olvb)

    dwx0 = dwx0_ref[...]
    dwh0 = dwh0_ref[...]
    db0 = db0_ref[...]
    dwc1 = dwc1_ref[...]
    db1 = db1_ref[...]

    def dec_body(j, carry):
        x2, h0d, c0d, h1d, c1d = carry
        h0n, c0n, h1n, c1n = lstm_step(x2, h0d, c0d, h1d, c1d,
                                       dwx0, dwh0, db0, dwc1, db1)
        o = head(h1n, ow, ob)
        olv = head(h1n, olvw, olvb)
        idx = _T - 2 - j
        out_ref[pl.ds(idx, 1)] = o[None]
        outlv_ref[pl.ds(idx, 1)] = olv[None]
        return o, h0n, c0n, h1n, c1n

    jax.lax.fori_loop(0, _T - 1, dec_body, (o_last, h0, c0, h1, c1))


def _pad_nodes(x, np_=_NP):
    pad = [(0, np_ - x.shape[0])] + [(0, 0)] * (x.ndim - 1)
    return jnp.pad(x, pad)


def kernel(ts_batch, edge_index, params):
    f32 = jnp.float32
    ts_k = jnp.transpose(ts_batch[..., 0], (1, 2, 0))
    ts_k = jnp.pad(ts_k, ((0, 0), (0, _NP - _N), (0, 0)))

    ei = edge_index.astype(jnp.int32)
    loops = jnp.arange(_N, dtype=jnp.int32)
    src_a = jnp.concatenate([ei[0], loops])
    dst_a = jnp.concatenate([ei[1], loops])
    padn = _EP - src_a.shape[0]
    src_col = jnp.pad(src_a, (0, padn), constant_values=-1).reshape(_EP, 1)
    dst_row = jnp.pad(dst_a, (0, padn), constant_values=-1).reshape(1, _EP)

    eps = jax.random.normal(jax.random.key(42), (_B, _N, _H), dtype=f32)
    eps_k = _pad_nodes(jnp.transpose(eps, (1, 0, 2)))

    def layer0_args(p):
        return (p["Wx"].astype(f32), p["Wh"].astype(jnp.bfloat16),
                (p["bx"] + p["bh"]).reshape(1, _G4).astype(f32))

    def layer1_args(p):
        wcat = jnp.concatenate([p["Wx"], p["Wh"]],
                               axis=0).astype(jnp.bfloat16)
        return wcat, (p["bx"] + p["bh"]).reshape(1, _G4).astype(f32)

    ewx0, ewh0, eb0 = layer0_args(params["enc"][0])
    ewc1, eb1 = layer1_args(params["enc"][1])
    dwx0, dwh0, db0 = layer0_args(params["dec"][0])
    dwc1, db1 = layer1_args(params["dec"][1])

    muw = _pad_nodes(params["mu_W"])
    mub = _pad_nodes(params["mu_b"])
    lvw = _pad_nodes(params["lv_W"])
    lvb = _pad_nodes(params["lv_b"])
    ow = _pad_nodes(params["out_W"][:, :, 0])
    ob = _pad_nodes(params["out_b"])
    olvw = _pad_nodes(params["outlv_W"][:, :, 0])
    olvb = _pad_nodes(params["outlv_b"])

    out_shapes = (
        jax.ShapeDtypeStruct((_T, _NP, _B), f32),
        jax.ShapeDtypeStruct((_T, _NP, _B), f32),
        jax.ShapeDtypeStruct((_NP, _B, _H), f32),
        jax.ShapeDtypeStruct((_NP, _B, _H), f32),
        jax.ShapeDtypeStruct((_NP, _B, _H), f32),
    )
    out_k, outlv_k, lat_k, mu_k, lv_k = pl.pallas_call(
        _fwd_kernel, out_shape=out_shapes)(
        ts_k, src_col, dst_row, eps_k,
        ewx0, ewh0, eb0, ewc1, eb1,
        dwx0, dwh0, db0, dwc1, db1,
        muw, mub, lvw, lvb, ow, ob, olvw, olvb)

    out = jnp.transpose(out_k[:, :_N, :], (2, 0, 1))[..., None]
    outlv = jnp.transpose(outlv_k[:, :_N, :], (2, 0, 1))[..., None]
    lat = jnp.transpose(lat_k[:_N], (1, 0, 2))
    mu = jnp.transpose(mu_k[:_N], (1, 0, 2))
    lv = jnp.transpose(lv_k[:_N], (1, 0, 2))
    return out, lat, mu, lv, outlv
